# baseline (speedup 1.0000x reference)
import functools

import jax
import jax.numpy as jnp
from jax import lax
from jax.experimental import pallas as pl
from jax.experimental.pallas import tpu as pltpu
from jax.experimental.pallas import tpu_sc as plsc

_VOCAB = 1000
_DIM = 64
_BATCH = 4096
_HIST = 200
_N = _BATCH * _HIST

_NC = 2
_NS = 16
_NW = _NC * _NS

_ROWS_PER_W = _N // _NW
_CHUNK = 400
_NCHUNK = _ROWS_PER_W // _CHUNK
_GROUPS = _CHUNK // 16

_mesh = plsc.VectorSubcoreMesh(core_axis_name="c", subcore_axis_name="s")


@functools.partial(
    pl.kernel,
    out_type=jax.ShapeDtypeStruct((_N * _DIM,), jnp.float32),
    mesh=_mesh,
    scratch_types=[
        pltpu.VMEM((_VOCAB * _DIM,), jnp.float32),
        pltpu.VMEM((_CHUNK,), jnp.int32),
        pltpu.VMEM((_CHUNK,), jnp.int32),
        pltpu.VMEM((_CHUNK * _DIM,), jnp.float32),
        pltpu.VMEM((_CHUNK * _DIM,), jnp.float32),
        pltpu.SemaphoreType.DMA,
        pltpu.SemaphoreType.DMA,
    ],
    compiler_params=pltpu.CompilerParams(use_tc_tiling_on_sc=False),
)
def _emb_lookup(idx_hbm, table_hbm, out_hbm,
                table_v, idx0, idx1, rows0, rows1, sem_w0, sem_w1):
    wid = lax.axis_index("s") * _NC + lax.axis_index("c")
    idx_base = wid * _ROWS_PER_W
    out_base = idx_base * _DIM

    pltpu.sync_copy(table_hbm, table_v)

    def load_idx(i, idx_v):
        pltpu.sync_copy(idx_hbm.at[pl.ds(idx_base + i * _CHUNK, _CHUNK)], idx_v)

    def gather_chunk(idx_v, rows_v):
        def group(g, carry):
            idxv = idx_v[pl.ds(g * 16, 16)]
            roff = g * (16 * _DIM)
            for l in range(16):
                b = idxv[l] * _DIM
                for k in range(_DIM // 16):
                    rows_v[pl.ds(roff + l * _DIM + k * 16, 16)] = (
                        table_v[pl.ds(b + k * 16, 16)])
            return carry
        lax.fori_loop(0, _GROUPS, group, 0)

    def fire_writeout(i, rows_v, sem):
        pltpu.async_copy(
            rows_v, out_hbm.at[pl.ds(out_base + i * _CHUNK * _DIM, _CHUNK * _DIM)],
            sem)

    def wait_writeout(i, rows_v, sem):
        pltpu.make_async_copy(
            rows_v, out_hbm.at[pl.ds(out_base + i * _CHUNK * _DIM, _CHUNK * _DIM)],
            sem).wait()

    def body(t, carry):
        a = 2 * t
        b = a + 1
        load_idx(a, idx0)

        @pl.when(t > 0)
        def _():
            wait_writeout(a - 2, rows0, sem_w0)

        gather_chunk(idx0, rows0)
        fire_writeout(a, rows0, sem_w0)
        load_idx(b, idx1)

        @pl.when(t > 0)
        def _():
            wait_writeout(b - 2, rows1, sem_w1)

        gather_chunk(idx1, rows1)
        fire_writeout(b, rows1, sem_w1)
        return carry

    lax.fori_loop(0, _NCHUNK // 2, body, 0)
    wait_writeout(_NCHUNK - 2, rows0, sem_w0)
    wait_writeout(_NCHUNK - 1, rows1, sem_w1)


def kernel(vocab_ids, table):
    idx = vocab_ids.reshape(_N).astype(jnp.int32)
    out = _emb_lookup(idx, table.reshape(_VOCAB * _DIM))
    return out.reshape(_BATCH, _HIST, _DIM)


# speedup vs baseline: 1.3080x; 1.3080x over previous
import functools

import jax
import jax.numpy as jnp
from jax import lax
from jax.experimental import pallas as pl
from jax.experimental.pallas import tpu as pltpu
from jax.experimental.pallas import tpu_sc as plsc

_VOCAB = 1000
_DIM = 64
_BATCH = 4096
_HIST = 200

_NC = 2
_NS = 16
_NW = _NC * _NS

_B_PER_W = _BATCH // _NW

_mesh = plsc.VectorSubcoreMesh(core_axis_name="c", subcore_axis_name="s")


@functools.partial(
    pl.kernel,
    out_type=jax.ShapeDtypeStruct((_BATCH, _HIST, _DIM), jnp.float32),
    mesh=_mesh,
    scratch_types=[
        pltpu.VMEM((512, 128), jnp.float32),
        pltpu.VMEM((8, _HIST), jnp.int32),
        pltpu.VMEM((_HIST, _DIM), jnp.float32),
        pltpu.VMEM((_HIST, _DIM), jnp.float32),
        pltpu.SemaphoreType.DMA,
        pltpu.SemaphoreType.DMA,
    ],
    compiler_params=pltpu.CompilerParams(use_tc_tiling_on_sc=True),
)
def _emb_lookup(idx_hbm, table_hbm, out_hbm,
                table_v, idx_v, rows0, rows1, sem_w0, sem_w1):
    wid = lax.axis_index("s") * _NC + lax.axis_index("c")
    b_base = wid * _B_PER_W

    pltpu.sync_copy(table_hbm, table_v)

    def gather_row(r, row_out):
        r2 = r // 2
        c0 = (r & 1) * _DIM
        for k in range(_DIM // 16):
            row_out[pl.ds(k * 16, 16)] = table_v[r2, pl.ds(c0 + k * 16, 16)]

    def gather_item(i, rows_v):
        def group(g, carry):
            idxv = idx_v[i, pl.ds(g * 16, 16)]
            for l in range(16):
                gather_row(idxv[l], rows_v.at[g * 16 + l])
            return carry
        lax.fori_loop(0, _HIST // 16, group, 0, unroll=False)
        idxv = idx_v[i, pl.ds(_HIST - 16, 16)]
        for l in range(8):
            gather_row(idxv[l + 8], rows_v.at[192 + l])

    def fire_writeout(b, rows_v, sem):
        pltpu.async_copy(rows_v, out_hbm.at[b_base + b], sem)

    def wait_writeout(b, rows_v, sem):
        pltpu.make_async_copy(rows_v, out_hbm.at[b_base + b], sem).wait()

    def body(t, carry):
        pltpu.sync_copy(idx_hbm.at[pl.ds(b_base + t * 8, 8)], idx_v)
        for i in range(0, 8, 2):
            a = t * 8 + i
            b = a + 1

            @pl.when(a > 0)
            def _():
                wait_writeout(a - 2, rows0, sem_w0)

            gather_item(i, rows0)
            fire_writeout(a, rows0, sem_w0)

            @pl.when(a > 0)
            def _():
                wait_writeout(b - 2, rows1, sem_w1)

            gather_item(i + 1, rows1)
            fire_writeout(b, rows1, sem_w1)
        return carry

    lax.fori_loop(0, _B_PER_W // 8, body, 0)
    wait_writeout(_B_PER_W - 2, rows0, sem_w0)
    wait_writeout(_B_PER_W - 1, rows1, sem_w1)


def kernel(vocab_ids, table):
    table_p = jnp.pad(table.reshape(_VOCAB // 2, 2 * _DIM),
                      ((0, 512 - _VOCAB // 2), (0, 0)))
    return _emb_lookup(vocab_ids.astype(jnp.int32), table_p)
